# baseline (speedup 1.0000x reference)
import functools

import jax
import jax.numpy as jnp
from jax import lax
from jax.experimental import pallas as pl
from jax.experimental.pallas import tpu as pltpu
from jax.experimental.pallas import tpu_sc as plsc

N = 10000
E = 160000
D_IN = 256
H1 = 8
C1 = 8
F = 64
TW = 128
DW = 128

NC = 2
NS = 16
NW = NC * NS
L = 16

B = 64
EP = E + N
NB_TOT = -(-EP // (NW * B)) * 2
E_PAD = NB_TOT * NS * B
NB_A = 96
NB_B = NB_TOT - NB_A
EPW_A = NB_A * B
EPW_B = NB_B * B
RPT = (-(-(N + 1) // NS) + 7) // 8 * 8
NROWS = RPT * NS
DUMMY = N


def _leaky(v):
    return jnp.where(v > 0, v, 0.2 * v)


def _prep1_body(x_ref, wa_ref, wb_ref, ts_ref, td_ref, mx_ref):
    xv = x_ref[...]
    ts = jnp.dot(xv, wa_ref[...], preferred_element_type=jnp.float32)
    td = jnp.dot(xv, wb_ref[...], preferred_element_type=jnp.float32)
    ts_ref[...] = ts
    td_ref[...] = td
    lm = _leaky(jnp.max(ts[:, F:]) + jnp.max(td))
    mx_ref[...] = jnp.full((8, 128), lm, jnp.float32)


def _prep1(x, wa, wb):
    return pl.pallas_call(
        _prep1_body,
        out_shape=[
            jax.ShapeDtypeStruct((N, TW), jnp.float32),
            jax.ShapeDtypeStruct((N, DW), jnp.float32),
            jax.ShapeDtypeStruct((8, 128), jnp.float32),
        ],
    )(x, wa, wb)


def _combine1_body(acc_ref, b1_ref, a2_ref, b2m_ref, ts_ref, td_ref, mx_ref):
    p = acc_ref[0, :N, :] + acc_ref[1, :N, :]
    msg = p[:, :F]
    den = p[:, F:]
    out1 = msg / (den + 1e-16) + b1_ref[...]
    h2 = jnp.where(out1 > 0, out1, jnp.exp(jnp.minimum(out1, 0.0)) - 1.0)
    ts = jnp.dot(h2, a2_ref[...], preferred_element_type=jnp.float32)
    td = jnp.dot(h2, b2m_ref[...], preferred_element_type=jnp.float32)
    ts_ref[...] = ts
    td_ref[...] = td
    lm = _leaky(jnp.max(ts[:, F:]) + jnp.max(td))
    mx_ref[...] = jnp.full((8, 128), lm, jnp.float32)


def _combine1(acc, b1, a2, b2m):
    return pl.pallas_call(
        _combine1_body,
        out_shape=[
            jax.ShapeDtypeStruct((N, TW), jnp.float32),
            jax.ShapeDtypeStruct((N, DW), jnp.float32),
            jax.ShapeDtypeStruct((8, 128), jnp.float32),
        ],
    )(acc, b1.reshape(1, F), a2, b2m)


def _combine2_body(acc_ref, b2_ref, out_ref):
    p = acc_ref[0, :N, :] + acc_ref[1, :N, :]
    msg = p[:, :F]
    den = p[:, F:]
    out_ref[...] = msg / (den + 1e-16) + b2_ref[...]


def _combine2(acc, b2):
    return pl.pallas_call(
        _combine2_body,
        out_shape=jax.ShapeDtypeStruct((N, F), jnp.float32),
    )(acc, b2.reshape(1, F))


@functools.lru_cache(maxsize=None)
def _make_edge_kernel():
    mesh = plsc.VectorSubcoreMesh(
        core_axis_name="c", subcore_axis_name="s",
        num_cores=NC, num_subcores=NS)

    def body(src_hbm, dst_hbm, tsrc_hbm, tdst_hbm, lm_hbm, zr_hbm, out_hbm,
             sidx0, sidx1, didx0, didx1, didx2, didx3,
             grows0, grows1, gdst0, gdst1, stage0, stage1, lmv, acc_sh,
             sis0, sis1, dis0, dis1, dis2, dis3,
             gsem0, gsem1, dsem0, dsem1, ssem0, ssem1, zsem):
        cid = lax.axis_index("c")
        sid = lax.axis_index("s")
        wbase = jnp.where(cid == 0, sid * EPW_A,
                          NS * EPW_A + sid * EPW_B)
        nb = jnp.where(cid == 0, NB_A, NB_B)
        pltpu.async_copy(zr_hbm, acc_sh.at[pl.ds(sid * RPT, RPT)], zsem)
        pltpu.sync_copy(lm_hbm, lmv)
        lmvec = lmv[...]
        sidx = (sidx0, sidx1)
        sis = (sis0, sis1)
        didx = (didx0, didx1, didx2, didx3)
        dis = (dis0, dis1, dis2, dis3)
        rows = ((grows0, gdst0, gsem0, dsem0),
                (grows1, gdst1, gsem1, dsem1))
        stage = (stage0, stage1)
        ssem = (ssem0, ssem1)

        def fire_idx(g, p, q):
            eb = pl.multiple_of(wbase + g * B, 8)
            pltpu.async_copy(src_hbm.at[pl.ds(eb, B)], sidx[p], sis[p])
            pltpu.async_copy(dst_hbm.at[pl.ds(eb, B)], didx[q], dis[q])

        def wait_idx(g, p, q):
            eb = pl.multiple_of(wbase + g * B, 8)
            pltpu.make_async_copy(
                src_hbm.at[pl.ds(eb, B)], sidx[p], sis[p]).wait()
            pltpu.make_async_copy(
                dst_hbm.at[pl.ds(eb, B)], didx[q], dis[q]).wait()

        def fire_rows(p, q):
            pltpu.async_copy(tsrc_hbm.at[sidx[p]], rows[p][0], rows[p][2])
            pltpu.async_copy(tdst_hbm.at[didx[q]], rows[p][1], rows[p][3])

        def wait_rows(p, q):
            pltpu.make_async_copy(
                tsrc_hbm.at[sidx[p]], rows[p][0], rows[p][2]).wait()
            pltpu.make_async_copy(
                tdst_hbm.at[didx[q]], rows[p][1], rows[p][3]).wait()

        def wait_scatter(p):
            pltpu.make_async_copy(
                stage[p], acc_sh.at[didx[p]], ssem[p]).wait()

        fire_idx(0, 0, 0)
        fire_idx(1, 1, 1)
        wait_idx(0, 0, 0)
        fire_rows(0, 0)
        pltpu.make_async_copy(
            zr_hbm, acc_sh.at[pl.ds(sid * RPT, RPT)], zsem).wait()
        plsc.subcore_barrier()

        def outer(g4, carry):
            for u in range(4):
                g = g4 * 4 + u
                p, q = u % 2, u
                gr, gd = rows[p][0], rows[p][1]

                @pl.when(g + 1 < nb)
                def _():
                    wait_idx(g + 1, 1 - p, (u + 1) % 4)

                wait_rows(p, q)

                @pl.when(g + 1 < nb)
                def _():
                    fire_rows(1 - p, (u + 1) % 4)

                @pl.when(g >= 2)
                def _():
                    wait_scatter(p)

                def edge(e4, c2):
                    for v in range(4):
                        e = e4 * 4 + v
                        for k in range(F // L):
                            o = k * L
                            a = gr[e, pl.ds(F + o, L)] + gd[e, pl.ds(o, L)]
                            phi = jnp.exp(_leaky(a) - lmvec)
                            stage[p][e, pl.ds(o, L)] = (
                                gr[e, pl.ds(o, L)] * phi)
                            stage[p][e, pl.ds(F + o, L)] = phi
                    return c2

                lax.fori_loop(0, B // 4, edge, 0)

                pltpu.async_copy(
                    stage[p], acc_sh.at[didx[q]], ssem[p], add=True)

                @pl.when(g + 2 < nb)
                def _():
                    fire_idx(g + 2, p, (u + 2) % 4)
            return carry

        lax.fori_loop(0, nb // 4, outer, 0)
        wait_scatter(0)
        wait_scatter(1)
        plsc.subcore_barrier()
        pltpu.sync_copy(
            acc_sh.at[pl.ds(sid * RPT, RPT)],
            out_hbm.at[pl.ds((cid * NS + sid) * RPT, RPT)])

    return pl.kernel(
        body,
        out_type=jax.ShapeDtypeStruct((NC * NROWS, TW), jnp.float32),
        mesh=mesh,
        scratch_types=[
            pltpu.VMEM((B,), jnp.int32),
            pltpu.VMEM((B,), jnp.int32),
            pltpu.VMEM((B,), jnp.int32),
            pltpu.VMEM((B,), jnp.int32),
            pltpu.VMEM((B,), jnp.int32),
            pltpu.VMEM((B,), jnp.int32),
            pltpu.VMEM((B, TW), jnp.float32),
            pltpu.VMEM((B, TW), jnp.float32),
            pltpu.VMEM((B, DW), jnp.float32),
            pltpu.VMEM((B, DW), jnp.float32),
            pltpu.VMEM((B, TW), jnp.float32),
            pltpu.VMEM((B, TW), jnp.float32),
            pltpu.VMEM((L,), jnp.float32),
            pltpu.VMEM_SHARED((NROWS, TW), jnp.float32),
        ] + [pltpu.SemaphoreType.DMA] * 13,
    )


def _expand_att(att):
    h = att.shape[0]
    return (att[:, :, None] * jnp.eye(h, dtype=att.dtype)[:, None, :]
            ).reshape(-1, h)


def _rep_mat(h):
    return (jnp.eye(h, dtype=jnp.float32)[:, :, None]
            * jnp.ones((1, 1, F // h), jnp.float32)).reshape(h, F)


@jax.jit
def kernel(x, edge_index, W1, att_src1, att_dst1, b1, W2, att_src2,
           att_dst2, b2):
    loop = jnp.arange(N, dtype=jnp.int32)
    npad = E_PAD - EP
    src = jnp.concatenate([
        edge_index[0].astype(jnp.int32), loop,
        jnp.zeros((npad,), jnp.int32)])
    dst = jnp.concatenate([
        edge_index[1].astype(jnp.int32), loop,
        jnp.full((npad,), DUMMY, jnp.int32)])

    eyef = jnp.eye(F, dtype=jnp.float32)
    rep1 = _rep_mat(H1)
    a1 = jnp.concatenate([eyef, _expand_att(att_src1) @ rep1], axis=1)
    b1m = jnp.concatenate(
        [_expand_att(att_dst1) @ rep1, jnp.zeros((F, F), jnp.float32)], axis=1)
    wa1 = W1 @ a1
    wb1 = W1 @ b1m
    rep2 = jnp.ones((1, F), jnp.float32)
    a2 = W2 @ jnp.concatenate([eyef, att_src2.T @ rep2], axis=1)
    b2m = W2 @ jnp.concatenate(
        [att_dst2.T @ rep2, jnp.zeros((F, F), jnp.float32)], axis=1)

    zr = jnp.zeros((RPT, TW), jnp.float32)
    edge_k = _make_edge_kernel()

    ts1, td1, mx1 = _prep1(x, wa1, wb1)
    td1p = jnp.pad(td1, ((0, NROWS - N), (0, 0)))
    lm1 = jnp.full((L,), mx1[0, 0], jnp.float32)
    acc1 = edge_k(src, dst, ts1, td1p, lm1, zr).reshape(NC, NROWS, TW)

    ts2, td2, mx2 = _combine1(acc1, b1, a2, b2m)
    td2p = jnp.pad(td2, ((0, NROWS - N), (0, 0)))
    lm2 = jnp.full((L,), mx2[0, 0], jnp.float32)
    acc2 = edge_k(src, dst, ts2, td2p, lm2, zr).reshape(NC, NROWS, TW)

    return _combine2(acc2, b2)


# speedup vs baseline: 1.0182x; 1.0182x over previous
import functools

import jax
import jax.numpy as jnp
from jax import lax
from jax.experimental import pallas as pl
from jax.experimental.pallas import tpu as pltpu
from jax.experimental.pallas import tpu_sc as plsc

N = 10000
E = 160000
D_IN = 256
H1 = 8
C1 = 8
F = 64
TW = 128
DW = 128

NC = 2
NS = 16
NW = NC * NS
L = 16

B = 64
EP = E + N
NB_TOT = -(-EP // (NW * B)) * 2
E_PAD = NB_TOT * NS * B
NB_A = 100
NB_B = NB_TOT - NB_A
EPW_A = NB_A * B
EPW_B = NB_B * B
RPT = (-(-(N + 1) // NS) + 7) // 8 * 8
NROWS = RPT * NS
DUMMY = N


def _leaky(v):
    return jnp.where(v > 0, v, 0.2 * v)


def _prep1_body(x_ref, wa_ref, wb_ref, ts_ref, td_ref, mx_ref):
    xv = x_ref[...]
    ts = jnp.dot(xv, wa_ref[...], preferred_element_type=jnp.float32)
    td = jnp.dot(xv, wb_ref[...], preferred_element_type=jnp.float32)
    ts_ref[...] = ts
    td_ref[...] = td
    lm = _leaky(jnp.max(ts[:, F:]) + jnp.max(td))
    mx_ref[...] = jnp.full((8, 128), lm, jnp.float32)


def _prep1(x, wa, wb):
    return pl.pallas_call(
        _prep1_body,
        out_shape=[
            jax.ShapeDtypeStruct((N, TW), jnp.float32),
            jax.ShapeDtypeStruct((N, DW), jnp.float32),
            jax.ShapeDtypeStruct((8, 128), jnp.float32),
        ],
    )(x, wa, wb)


def _combine1_body(acc_ref, b1_ref, a2_ref, b2m_ref, ts_ref, td_ref, mx_ref):
    p = acc_ref[0, :N, :] + acc_ref[1, :N, :]
    msg = p[:, :F]
    den = p[:, F:]
    out1 = msg / (den + 1e-16) + b1_ref[...]
    h2 = jnp.where(out1 > 0, out1, jnp.exp(jnp.minimum(out1, 0.0)) - 1.0)
    ts = jnp.dot(h2, a2_ref[...], preferred_element_type=jnp.float32)
    td = jnp.dot(h2, b2m_ref[...], preferred_element_type=jnp.float32)
    ts_ref[...] = ts
    td_ref[...] = td
    lm = _leaky(jnp.max(ts[:, F:]) + jnp.max(td))
    mx_ref[...] = jnp.full((8, 128), lm, jnp.float32)


def _combine1(acc, b1, a2, b2m):
    return pl.pallas_call(
        _combine1_body,
        out_shape=[
            jax.ShapeDtypeStruct((N, TW), jnp.float32),
            jax.ShapeDtypeStruct((N, DW), jnp.float32),
            jax.ShapeDtypeStruct((8, 128), jnp.float32),
        ],
    )(acc, b1.reshape(1, F), a2, b2m)


def _combine2_body(acc_ref, b2_ref, out_ref):
    p = acc_ref[0, :N, :] + acc_ref[1, :N, :]
    msg = p[:, :F]
    den = p[:, F:]
    out_ref[...] = msg / (den + 1e-16) + b2_ref[...]


def _combine2(acc, b2):
    return pl.pallas_call(
        _combine2_body,
        out_shape=jax.ShapeDtypeStruct((N, F), jnp.float32),
    )(acc, b2.reshape(1, F))


@functools.lru_cache(maxsize=None)
def _make_edge_kernel():
    mesh = plsc.VectorSubcoreMesh(
        core_axis_name="c", subcore_axis_name="s",
        num_cores=NC, num_subcores=NS)

    def body(src_hbm, dst_hbm, tsrc_hbm, tdst_hbm, lm_hbm, zr_hbm, out_hbm,
             sidx0, sidx1, didx0, didx1, didx2, didx3,
             grows0, grows1, gdst0, gdst1, stage0, stage1, lmv, acc_sh,
             sis0, sis1, dis0, dis1, dis2, dis3,
             gsem0, gsem1, dsem0, dsem1, ssem0, ssem1, zsem):
        cid = lax.axis_index("c")
        sid = lax.axis_index("s")
        wbase = jnp.where(cid == 0, sid * EPW_A,
                          NS * EPW_A + sid * EPW_B)
        nb = jnp.where(cid == 0, NB_A, NB_B)
        pltpu.async_copy(zr_hbm, acc_sh.at[pl.ds(sid * RPT, RPT)], zsem)
        pltpu.sync_copy(lm_hbm, lmv)
        lmvec = lmv[...]
        sidx = (sidx0, sidx1)
        sis = (sis0, sis1)
        didx = (didx0, didx1, didx2, didx3)
        dis = (dis0, dis1, dis2, dis3)
        rows = ((grows0, gdst0, gsem0, dsem0),
                (grows1, gdst1, gsem1, dsem1))
        stage = (stage0, stage1)
        ssem = (ssem0, ssem1)

        def fire_idx(g, p, q):
            eb = pl.multiple_of(wbase + g * B, 8)
            pltpu.async_copy(src_hbm.at[pl.ds(eb, B)], sidx[p], sis[p])
            pltpu.async_copy(dst_hbm.at[pl.ds(eb, B)], didx[q], dis[q])

        def wait_idx(g, p, q):
            eb = pl.multiple_of(wbase + g * B, 8)
            pltpu.make_async_copy(
                src_hbm.at[pl.ds(eb, B)], sidx[p], sis[p]).wait()
            pltpu.make_async_copy(
                dst_hbm.at[pl.ds(eb, B)], didx[q], dis[q]).wait()

        def fire_rows(p, q):
            pltpu.async_copy(tsrc_hbm.at[sidx[p]], rows[p][0], rows[p][2])
            pltpu.async_copy(tdst_hbm.at[didx[q]], rows[p][1], rows[p][3])

        def wait_rows(p, q):
            pltpu.make_async_copy(
                tsrc_hbm.at[sidx[p]], rows[p][0], rows[p][2]).wait()
            pltpu.make_async_copy(
                tdst_hbm.at[didx[q]], rows[p][1], rows[p][3]).wait()

        def wait_scatter(p):
            pltpu.make_async_copy(
                stage[p], acc_sh.at[didx[p]], ssem[p]).wait()

        fire_idx(0, 0, 0)
        fire_idx(1, 1, 1)
        wait_idx(0, 0, 0)
        fire_rows(0, 0)
        pltpu.make_async_copy(
            zr_hbm, acc_sh.at[pl.ds(sid * RPT, RPT)], zsem).wait()
        plsc.subcore_barrier()

        def outer(g4, carry):
            for u in range(4):
                g = g4 * 4 + u
                p, q = u % 2, u
                gr, gd = rows[p][0], rows[p][1]

                @pl.when(g + 1 < nb)
                def _():
                    wait_idx(g + 1, 1 - p, (u + 1) % 4)

                wait_rows(p, q)

                @pl.when(g + 1 < nb)
                def _():
                    fire_rows(1 - p, (u + 1) % 4)

                @pl.when(g >= 2)
                def _():
                    wait_scatter(p)

                def edge(e4, c2):
                    for v in range(4):
                        e = e4 * 4 + v
                        for k in range(F // L):
                            o = k * L
                            a = gr[e, pl.ds(F + o, L)] + gd[e, pl.ds(o, L)]
                            phi = jnp.exp(_leaky(a) - lmvec)
                            stage[p][e, pl.ds(o, L)] = (
                                gr[e, pl.ds(o, L)] * phi)
                            stage[p][e, pl.ds(F + o, L)] = phi
                    return c2

                lax.fori_loop(0, B // 4, edge, 0)

                pltpu.async_copy(
                    stage[p], acc_sh.at[didx[q]], ssem[p], add=True)

                @pl.when(g + 2 < nb)
                def _():
                    fire_idx(g + 2, p, (u + 2) % 4)
            return carry

        lax.fori_loop(0, nb // 4, outer, 0)
        wait_scatter(0)
        wait_scatter(1)
        plsc.subcore_barrier()
        pltpu.sync_copy(
            acc_sh.at[pl.ds(sid * RPT, RPT)],
            out_hbm.at[pl.ds((cid * NS + sid) * RPT, RPT)])

    return pl.kernel(
        body,
        out_type=jax.ShapeDtypeStruct((NC * NROWS, TW), jnp.float32),
        mesh=mesh,
        scratch_types=[
            pltpu.VMEM((B,), jnp.int32),
            pltpu.VMEM((B,), jnp.int32),
            pltpu.VMEM((B,), jnp.int32),
            pltpu.VMEM((B,), jnp.int32),
            pltpu.VMEM((B,), jnp.int32),
            pltpu.VMEM((B,), jnp.int32),
            pltpu.VMEM((B, TW), jnp.float32),
            pltpu.VMEM((B, TW), jnp.float32),
            pltpu.VMEM((B, DW), jnp.float32),
            pltpu.VMEM((B, DW), jnp.float32),
            pltpu.VMEM((B, TW), jnp.float32),
            pltpu.VMEM((B, TW), jnp.float32),
            pltpu.VMEM((L,), jnp.float32),
            pltpu.VMEM_SHARED((NROWS, TW), jnp.float32),
        ] + [pltpu.SemaphoreType.DMA] * 13,
    )


def _expand_att(att):
    h = att.shape[0]
    return (att[:, :, None] * jnp.eye(h, dtype=att.dtype)[:, None, :]
            ).reshape(-1, h)


def _rep_mat(h):
    return (jnp.eye(h, dtype=jnp.float32)[:, :, None]
            * jnp.ones((1, 1, F // h), jnp.float32)).reshape(h, F)


@jax.jit
def kernel(x, edge_index, W1, att_src1, att_dst1, b1, W2, att_src2,
           att_dst2, b2):
    loop = jnp.arange(N, dtype=jnp.int32)
    npad = E_PAD - EP
    src = jnp.concatenate([
        edge_index[0].astype(jnp.int32), loop,
        jnp.zeros((npad,), jnp.int32)])
    dst = jnp.concatenate([
        edge_index[1].astype(jnp.int32), loop,
        jnp.full((npad,), DUMMY, jnp.int32)])

    eyef = jnp.eye(F, dtype=jnp.float32)
    rep1 = _rep_mat(H1)
    a1 = jnp.concatenate([eyef, _expand_att(att_src1) @ rep1], axis=1)
    b1m = jnp.concatenate(
        [_expand_att(att_dst1) @ rep1, jnp.zeros((F, F), jnp.float32)], axis=1)
    wa1 = W1 @ a1
    wb1 = W1 @ b1m
    rep2 = jnp.ones((1, F), jnp.float32)
    a2 = W2 @ jnp.concatenate([eyef, att_src2.T @ rep2], axis=1)
    b2m = W2 @ jnp.concatenate(
        [att_dst2.T @ rep2, jnp.zeros((F, F), jnp.float32)], axis=1)

    zr = jnp.zeros((RPT, TW), jnp.float32)
    edge_k = _make_edge_kernel()

    ts1, td1, mx1 = _prep1(x, wa1, wb1)
    td1p = jnp.pad(td1, ((0, NROWS - N), (0, 0)))
    lm1 = jnp.full((L,), mx1[0, 0], jnp.float32)
    acc1 = edge_k(src, dst, ts1, td1p, lm1, zr).reshape(NC, NROWS, TW)

    ts2, td2, mx2 = _combine1(acc1, b1, a2, b2m)
    td2p = jnp.pad(td2, ((0, NROWS - N), (0, 0)))
    lm2 = jnp.full((L,), mx2[0, 0], jnp.float32)
    acc2 = edge_k(src, dst, ts2, td2p, lm2, zr).reshape(NC, NROWS, TW)

    return _combine2(acc2, b2)
